# baseline (speedup 1.0000x reference)
import functools

import jax
import jax.numpy as jnp
from jax import lax
from jax.experimental import pallas as pl
from jax.experimental.pallas import tpu as pltpu
from jax.experimental.pallas import tpu_sc as plsc

_B, _H, _W = 4, 224, 224
_S = 196
_Q = 49
_K = 9
_RS = 8
_NST = _H // _RS
_NUNIT = _B * _NST
_UPW = _NUNIT // 16


def _body(sims_hbm, sind_hbm, out_hbm):
    wid = lax.axis_index("s") * 2 + lax.axis_index("c")
    lane16 = wid & 15
    iota = lax.iota(jnp.int32, 16)

    def make_runner(w0, ncol):
        def scoped(bufA, bufB, sind_v, out_v, semA, semB, semO, semI):
            bufs = (bufA, bufB)
            sems = (semA, semB)
            civs = [iota + c0 for c0 in range(0, ncol, 16)]

            def decode(t):
                ust = lane16 * _UPW + t
                b = ust // _NST
                st = ust % _NST
                return b, st * _RS

            def slab_src(b, h0, q):
                return sims_hbm.at[b, pl.ds(q * _Q, _Q),
                                   pl.ds(h0, _RS), pl.ds(w0, ncol)]

            def out_dst(b, h0):
                return out_hbm.at[b, :, pl.ds(h0, _RS), pl.ds(w0, ncol)]

            def sind_src(b, h0):
                return sind_hbm.at[b, :, pl.ds(h0, _RS), pl.ds(w0, ncol)]

            b0, h00 = decode(0)
            pltpu.async_copy(slab_src(b0, h00, 0), bufA, semA)
            pltpu.async_copy(slab_src(b0, h00, 1), bufB, semB)
            pltpu.async_copy(sind_src(b0, h00), sind_v.at[0], semI)

            def unit_body(t, carry):
                b, h0 = decode(t)
                tn = jnp.minimum(t + 1, _UPW - 1)
                bn, h0n = decode(tn)
                par = t & 1
                for p in range(4):
                    buf, sem = bufs[p & 1], sems[p & 1]
                    pltpu.make_async_copy(
                        slab_src(b, h0, p), buf, sem).wait()
                    if p == 0:
                        @pl.when(t > 0)
                        def _():
                            tp = jnp.maximum(t - 1, 0)
                            bp, h0p = decode(tp)
                            pltpu.make_async_copy(
                                out_v, out_dst(bp, h0p), semO).wait()
                        pltpu.make_async_copy(
                            sind_src(b, h0), sind_v.at[par], semI).wait()

                    def kr_body(i, c2):
                        k = i >> 3
                        r = i & 7
                        rvec = jnp.full((16,), r, jnp.int32)
                        for ci in range(ncol // 16):
                            sl = (k, r, pl.ds(ci * 16, 16))
                            sv = sind_v[(par,) + sl]
                            if p == 0:
                                m = sv < _Q
                                out_v[sl] = plsc.load_gather(
                                    buf, [sv, rvec, civs[ci]], mask=m)
                            else:
                                loc = sv - (p * _Q)
                                m = (loc >= 0) & (loc < _Q)
                                g = plsc.load_gather(
                                    buf, [loc, rvec, civs[ci]], mask=m)
                                plsc.addupdate(out_v.at[sl], g)
                        return c2

                    lax.fori_loop(0, _K * _RS, kr_body, 0)
                    if p < 2:
                        pltpu.async_copy(slab_src(b, h0, p + 2), buf, sem)
                    else:
                        @pl.when(t + 1 < _UPW)
                        def _():
                            pltpu.async_copy(
                                slab_src(bn, h0n, p - 2), buf, sem)
                            if p == 2:
                                pltpu.async_copy(
                                    sind_src(bn, h0n),
                                    sind_v.at[(t + 1) & 1], semI)
                pltpu.async_copy(out_v, out_dst(b, h0), semO)
                return carry

            lax.fori_loop(0, _UPW, unit_body, 0)
            bl, h0l = decode(_UPW - 1)
            pltpu.make_async_copy(out_v, out_dst(bl, h0l), semO).wait()

        return scoped

    @pl.when(wid < 16)
    def _():
        pl.run_scoped(
            make_runner(0, 128),
            pltpu.VMEM((_Q, _RS, 128), jnp.float32),
            pltpu.VMEM((_Q, _RS, 128), jnp.float32),
            pltpu.VMEM((2, _K, _RS, 128), jnp.int32),
            pltpu.VMEM((_K, _RS, 128), jnp.float32),
            pltpu.SemaphoreType.DMA,
            pltpu.SemaphoreType.DMA,
            pltpu.SemaphoreType.DMA,
            pltpu.SemaphoreType.DMA,
        )

    @pl.when(wid >= 16)
    def _():
        pl.run_scoped(
            make_runner(128, _W - 128),
            pltpu.VMEM((_Q, _RS, _W - 128), jnp.float32),
            pltpu.VMEM((_Q, _RS, _W - 128), jnp.float32),
            pltpu.VMEM((2, _K, _RS, _W - 128), jnp.int32),
            pltpu.VMEM((_K, _RS, _W - 128), jnp.float32),
            pltpu.SemaphoreType.DMA,
            pltpu.SemaphoreType.DMA,
            pltpu.SemaphoreType.DMA,
            pltpu.SemaphoreType.DMA,
        )


@functools.partial(
    pl.kernel,
    out_type=jax.ShapeDtypeStruct((_B, _K, _H, _W), jnp.float32),
    mesh=plsc.VectorSubcoreMesh(core_axis_name="c", subcore_axis_name="s"),
    compiler_params=pltpu.CompilerParams(needs_layout_passes=False),
)
def _gather_sims_sc(sims_hbm, sind_hbm, out_hbm):
    _body(sims_hbm, sind_hbm, out_hbm)


def kernel(sims, sinds):
    b, h, w, sh, sw = sims.shape
    k = sinds.shape[-1]
    sims_t = jnp.transpose(sims, (0, 3, 4, 1, 2)).reshape(b, sh * sw, h, w)
    sind_t = jnp.transpose(sinds.astype(jnp.int32), (0, 3, 1, 2))
    out_t = _gather_sims_sc(sims_t, sind_t)
    return jnp.transpose(out_t, (0, 2, 3, 1))


# speedup vs baseline: 1.0207x; 1.0207x over previous
import functools

import jax
import jax.numpy as jnp
from jax import lax
from jax.experimental import pallas as pl
from jax.experimental.pallas import tpu as pltpu
from jax.experimental.pallas import tpu_sc as plsc

_B, _H, _W = 4, 224, 224
_S = 196
_Q = 49
_K = 9
_RS = 8
_NST = _H // _RS
_NUNIT = _B * _NST
_UPW = _NUNIT // 16


def _body(sims_hbm, sind_hbm, out_hbm):
    wid = lax.axis_index("s") * 2 + lax.axis_index("c")
    lane16 = wid & 15
    iota = lax.iota(jnp.int32, 16)

    def make_runner(w0, ncol):
        def scoped(bufA, bufB, sind_v, out_v, semA, semB, semO):
            bufs = (bufA, bufB)
            sems = (semA, semB)
            civs = [iota + c0 for c0 in range(0, ncol, 16)]

            def decode(t):
                ust = lane16 * _UPW + t
                b = ust // _NST
                st = ust % _NST
                return b, st * _RS

            def slab_src(b, h0, q):
                return sims_hbm.at[b, pl.ds(q * _Q, _Q),
                                   pl.ds(h0, _RS), pl.ds(w0, ncol)]

            def out_dst(b, h0):
                return out_hbm.at[b, :, pl.ds(h0, _RS), pl.ds(w0, ncol)]

            b0, h00 = decode(0)
            pltpu.async_copy(slab_src(b0, h00, 0), bufA, semA)
            pltpu.async_copy(slab_src(b0, h00, 1), bufB, semB)

            def unit_body(t, carry):
                b, h0 = decode(t)
                tn = jnp.minimum(t + 1, _UPW - 1)
                bn, h0n = decode(tn)
                pltpu.sync_copy(
                    sind_hbm.at[b, :, pl.ds(h0, _RS), pl.ds(w0, ncol)],
                    sind_v)
                for p in range(4):
                    buf, sem = bufs[p & 1], sems[p & 1]
                    pltpu.make_async_copy(
                        slab_src(b, h0, p), buf, sem).wait()
                    if p == 0:
                        @pl.when(t > 0)
                        def _():
                            tp = jnp.maximum(t - 1, 0)
                            bp, h0p = decode(tp)
                            pltpu.make_async_copy(
                                out_v, out_dst(bp, h0p), semO).wait()

                    def kr_body(i, c2):
                        k = i >> 3
                        r = i & 7
                        rvec = jnp.full((16,), r, jnp.int32)
                        for ci in range(ncol // 16):
                            sl = (k, r, pl.ds(ci * 16, 16))
                            sv = sind_v[sl]
                            if p == 0:
                                m = sv < _Q
                                out_v[sl] = plsc.load_gather(
                                    buf, [sv, rvec, civs[ci]], mask=m)
                            else:
                                loc = sv - (p * _Q)
                                m = (loc >= 0) & (loc < _Q)
                                g = plsc.load_gather(
                                    buf, [loc, rvec, civs[ci]], mask=m)
                                plsc.addupdate(out_v.at[sl], g)
                        return c2

                    lax.fori_loop(0, _K * _RS, kr_body, 0)
                    if p < 2:
                        pltpu.async_copy(slab_src(b, h0, p + 2), buf, sem)
                    else:
                        @pl.when(t + 1 < _UPW)
                        def _():
                            pltpu.async_copy(
                                slab_src(bn, h0n, p - 2), buf, sem)
                pltpu.async_copy(out_v, out_dst(b, h0), semO)
                return carry

            lax.fori_loop(0, _UPW, unit_body, 0)
            bl, h0l = decode(_UPW - 1)
            pltpu.make_async_copy(out_v, out_dst(bl, h0l), semO).wait()

        return scoped

    @pl.when(wid < 16)
    def _():
        pl.run_scoped(
            make_runner(0, 128),
            pltpu.VMEM((_Q, _RS, 128), jnp.float32),
            pltpu.VMEM((_Q, _RS, 128), jnp.float32),
            pltpu.VMEM((_K, _RS, 128), jnp.int32),
            pltpu.VMEM((_K, _RS, 128), jnp.float32),
            pltpu.SemaphoreType.DMA,
            pltpu.SemaphoreType.DMA,
            pltpu.SemaphoreType.DMA,
        )

    @pl.when(wid >= 16)
    def _():
        pl.run_scoped(
            make_runner(128, _W - 128),
            pltpu.VMEM((_Q, _RS, _W - 128), jnp.float32),
            pltpu.VMEM((_Q, _RS, _W - 128), jnp.float32),
            pltpu.VMEM((_K, _RS, _W - 128), jnp.int32),
            pltpu.VMEM((_K, _RS, _W - 128), jnp.float32),
            pltpu.SemaphoreType.DMA,
            pltpu.SemaphoreType.DMA,
            pltpu.SemaphoreType.DMA,
        )


@functools.partial(
    pl.kernel,
    out_type=jax.ShapeDtypeStruct((_B, _K, _H, _W), jnp.float32),
    mesh=plsc.VectorSubcoreMesh(core_axis_name="c", subcore_axis_name="s"),
    compiler_params=pltpu.CompilerParams(needs_layout_passes=False),
)
def _gather_sims_sc(sims_hbm, sind_hbm, out_hbm):
    _body(sims_hbm, sind_hbm, out_hbm)


def kernel(sims, sinds):
    b, h, w, sh, sw = sims.shape
    k = sinds.shape[-1]
    sims_t = jnp.transpose(sims, (0, 3, 4, 1, 2)).reshape(b, sh * sw, h, w)
    sind_t = jnp.transpose(sinds.astype(jnp.int32), (0, 3, 1, 2))
    out_t = _gather_sims_sc(sims_t, sind_t)
    return jnp.transpose(out_t, (0, 2, 3, 1))
